# baseline (speedup 1.0000x reference)
import functools

import jax
import jax.numpy as jnp
from jax import lax
from jax.experimental import pallas as pl
from jax.experimental.pallas import tpu as pltpu
from jax.experimental.pallas import tpu_sc as plsc

_NC = 2
_NS = 16
_LANES = 16
_CHUNK = 100
_NBUF = 4
_ZBLK = 64


_SC_PARAMS = pltpu.CompilerParams(use_tc_tiling_on_sc=False,
                                  needs_layout_passes=False)


def _sc_gcn(h_pad, src_w, dst_w, *, n_pad, cw):
    R = n_pad // _NS

    mesh = plsc.VectorSubcoreMesh(
        core_axis_name="c", subcore_axis_name="s",
        num_cores=_NC, num_subcores=_NS)

    @functools.partial(
        pl.kernel,
        out_type=jax.ShapeDtypeStruct((_NC, n_pad, _LANES), jnp.float32),
        mesh=mesh,
        compiler_params=_SC_PARAMS,
        scratch_types=[
            pltpu.VMEM_SHARED((n_pad,), jnp.float32),
            pltpu.VMEM_SHARED((n_pad, _LANES), jnp.float32),
            pltpu.VMEM_SHARED((n_pad, _LANES), jnp.float32),
            pltpu.VMEM((cw, _CHUNK), jnp.int32),
            pltpu.VMEM((cw, _CHUNK), jnp.int32),
            [pltpu.VMEM((_CHUNK, _LANES), jnp.float32)
             for _ in range(_NBUF)],
            pltpu.VMEM((112,), jnp.float32),
            pltpu.VMEM((R,), jnp.float32),
            pltpu.VMEM((R,), jnp.float32),
            pltpu.VMEM((R, _LANES), jnp.float32),
            pltpu.VMEM((R, _LANES), jnp.float32),
            pltpu.VMEM((_ZBLK, _LANES), jnp.float32),
            pltpu.SemaphoreType.DMA,
        ],
    )
    def k(h_hbm, src_hbm, dst_hbm, y_hbm,
          deg_sh, hp_sh, acc_sh, src_v, dst_v, rows, ones_v,
          degf, disf, hv, hpv, zerov, sem):
        c = lax.axis_index("c")
        s = lax.axis_index("s")
        w = c * _NS + s
        row0 = s * R

        fzero = jnp.zeros((_LANES,), jnp.float32)
        fone = jnp.full((_LANES,), 1.0, jnp.float32)
        half = jnp.full((_LANES,), 0.5, jnp.float32)
        three_half = jnp.full((_LANES,), 1.5, jnp.float32)
        magic = jnp.full((_LANES,), 0x5F3759DF, jnp.int32)
        one_i = jnp.full((_LANES,), 1, jnp.int32)

        def zfill_body(i, _):
            zerov[i] = fzero
            return 0
        lax.fori_loop(0, _ZBLK, zfill_body, 0)

        def ones_body(i, _):
            ones_v[pl.ds(i * _LANES, _LANES)] = fone
            return 0
        lax.fori_loop(0, 112 // _LANES, ones_body, 0)

        def degz_body(i, _):
            degf[pl.ds(i * _LANES, _LANES)] = fzero
            return 0
        lax.fori_loop(0, R // _LANES, degz_body, 0)
        pltpu.sync_copy(degf, deg_sh.at[pl.ds(row0, R)])

        def zero_acc(i, _):
            pltpu.sync_copy(zerov, acc_sh.at[pl.ds(row0 + i * _ZBLK, _ZBLK)])
            return 0
        lax.fori_loop(0, R // _ZBLK, zero_acc, 0)
        plsc.subcore_barrier()

        def deg_pass(dw):
            pltpu.sync_copy(dst_hbm.at[dw], dst_v)

            def deg_body(g, _):
                descs = [
                    pltpu.async_copy(
                        ones_v.at[pl.ds(0, _CHUNK)],
                        deg_sh.at[dst_v.at[_NBUF * g + b]], sem,
                        add=True)
                    for b in range(_NBUF)
                ]
                for dsc in descs:
                    dsc.wait()
                return 0
            lax.fori_loop(0, cw // _NBUF, deg_body, 0)

        deg_pass(2 * s)
        deg_pass(2 * s + 1)
        plsc.subcore_barrier()

        pltpu.sync_copy(deg_sh.at[pl.ds(row0, R)], degf)
        pltpu.sync_copy(h_hbm.at[pl.ds(row0, R)], hv)

        def rs_body(i, _):
            d = degf[pl.ds(i * _LANES, _LANES)] + fone
            bits = plsc.bitcast(d, jnp.int32)
            y = plsc.bitcast(
                magic - lax.shift_right_arithmetic(bits, one_i), jnp.float32)
            hd = half * d
            y = y * (three_half - hd * y * y)
            y = y * (three_half - hd * y * y)
            y = y * (three_half - hd * y * y)
            disf[pl.ds(i * _LANES, _LANES)] = y
            return 0
        lax.fori_loop(0, R // _LANES, rs_body, 0)

        def hp_body(i, _):
            dsp = plsc.load_gather(disf, [jnp.full((_LANES,), 1, jnp.int32) * i])
            hpv[i] = hv[i] * dsp
            return 0
        lax.fori_loop(0, R, hp_body, 0)
        pltpu.sync_copy(hpv, hp_sh.at[pl.ds(row0, R)])

        pltpu.sync_copy(src_hbm.at[w], src_v)
        pltpu.sync_copy(dst_hbm.at[w], dst_v)
        plsc.subcore_barrier()

        def edge_body(g, _):
            descs = [
                pltpu.async_copy(
                    hp_sh.at[src_v.at[_NBUF * g + b]], rows[b], sem)
                for b in range(_NBUF)
            ]
            for b in range(_NBUF):
                descs[b].wait()
                pltpu.sync_copy(
                    rows[b], acc_sh.at[dst_v.at[_NBUF * g + b]], add=True)
            return 0
        lax.fori_loop(0, cw // _NBUF, edge_body, 0)
        plsc.subcore_barrier()

        pltpu.sync_copy(acc_sh.at[pl.ds(row0, R)], hv)

        def y_body(i, _):
            dsp = plsc.load_gather(disf, [jnp.full((_LANES,), 1, jnp.int32) * i])
            hpv[i] = dsp * (hv[i] + half * hpv[i])
            return 0
        lax.fori_loop(0, R, y_body, 0)
        pltpu.sync_copy(hpv, y_hbm.at[c, pl.ds(row0, R)])

    return k(h_pad, src_w, dst_w)


def _tc_linear(x, w1, b1, *, n, n_pad, h):
    blk = 256
    grid = n_pad // blk

    def body(x_ref, w_ref, b_ref, o_ref):
        i = pl.program_id(0)
        acc = jnp.dot(x_ref[...], w_ref[...],
                      preferred_element_type=jnp.float32) + b_ref[...]
        rows = i * blk + lax.broadcasted_iota(jnp.int32, (blk, h), 0)
        o_ref[...] = jnp.where(rows < n, acc, 0.0)

    d = x.shape[1]
    return pl.pallas_call(
        body,
        grid=(grid,),
        in_specs=[
            pl.BlockSpec((blk, d), lambda i: (i, 0)),
            pl.BlockSpec((d, h), lambda i: (0, 0)),
            pl.BlockSpec((1, h), lambda i: (0, 0)),
        ],
        out_specs=pl.BlockSpec((blk, h), lambda i: (i, 0)),
        out_shape=jax.ShapeDtypeStruct((n_pad, h), jnp.float32),
    )(x, w1, b1.reshape(1, h))


def _tc_head(y0, y1, w2, b2, *, n, h, c):
    blk = 400
    grid = -(-n // blk)

    def body(a_ref, b_ref, w_ref, bias_ref, o_ref):
        z = jnp.maximum(a_ref[...] + b_ref[...], 0.0)
        logits = jnp.dot(z, w_ref[...],
                         preferred_element_type=jnp.float32) + bias_ref[...]
        m = jnp.max(logits, axis=1, keepdims=True)
        e = jnp.exp(logits - m)
        o_ref[...] = e / jnp.sum(e, axis=1, keepdims=True)

    rows = pl.BlockSpec((blk, h), lambda i: (i, 0))
    return pl.pallas_call(
        body,
        grid=(grid,),
        in_specs=[
            rows, rows,
            pl.BlockSpec((h, c), lambda i: (0, 0)),
            pl.BlockSpec((1, c), lambda i: (0, 0)),
        ],
        out_specs=pl.BlockSpec((blk, c), lambda i: (i, 0)),
        out_shape=jax.ShapeDtypeStruct((n, c), jnp.float32),
    )(y0, y1, w2, b2.reshape(1, c))


def kernel(x, edge_index, W1, b1, W2, b2):
    n, d = x.shape
    h = W1.shape[1]
    c = W2.shape[1]
    e = edge_index.shape[1]

    n_pad = -(-(n + 64) // 256) * 256
    junk = n_pad - n
    epw = _NC * _NS * _CHUNK * _NBUF
    e_pad = -(-e // epw) * epw
    cw = e_pad // (_NC * _NS * _CHUNK)

    h_pad = _tc_linear(x, W1, b1, n=n, n_pad=n_pad, h=h)

    src = edge_index[0]
    dst = edge_index[1]
    pad_cnt = e_pad - e
    if pad_cnt:
        pad_idx = n + jnp.arange(pad_cnt, dtype=jnp.int32) % junk
        src = jnp.concatenate([src, pad_idx])
        dst = jnp.concatenate([dst, pad_idx])
    src_w = src.reshape(_NC * _NS, cw, _CHUNK)
    dst_w = dst.reshape(_NC * _NS, cw, _CHUNK)

    y = _sc_gcn(h_pad, src_w, dst_w, n_pad=n_pad, cw=cw)
    return _tc_head(y[0], y[1], W2, b2, n=n, h=h, c=c)


# speedup vs baseline: 1.2734x; 1.2734x over previous
import functools

import jax
import jax.numpy as jnp
from jax import lax
from jax.experimental import pallas as pl
from jax.experimental.pallas import tpu as pltpu
from jax.experimental.pallas import tpu_sc as plsc

_NC = 2
_NS = 16
_LANES = 16
_CHUNK = 100
_NBUF = 4
_ZBLK = 64


_SC_PARAMS = pltpu.CompilerParams(use_tc_tiling_on_sc=False,
                                  needs_layout_passes=False)


def _sc_gcn(h_pad, src_w, dst_w, *, n_pad, cw):
    R = n_pad // _NS

    mesh = plsc.VectorSubcoreMesh(
        core_axis_name="c", subcore_axis_name="s",
        num_cores=_NC, num_subcores=_NS)

    @functools.partial(
        pl.kernel,
        out_type=jax.ShapeDtypeStruct((_NC, n_pad, _LANES), jnp.float32),
        mesh=mesh,
        compiler_params=_SC_PARAMS,
        scratch_types=[
            pltpu.VMEM_SHARED((n_pad,), jnp.float32),
            pltpu.VMEM_SHARED((n_pad, _LANES), jnp.float32),
            pltpu.VMEM_SHARED((n_pad, _LANES), jnp.float32),
            pltpu.VMEM((cw, _CHUNK), jnp.int32),
            pltpu.VMEM((cw, _CHUNK), jnp.int32),
            [pltpu.VMEM((_CHUNK, _LANES), jnp.float32)
             for _ in range(_NBUF)],
            pltpu.VMEM((112,), jnp.float32),
            pltpu.VMEM((R,), jnp.float32),
            pltpu.VMEM((R,), jnp.float32),
            pltpu.VMEM((R, _LANES), jnp.float32),
            pltpu.VMEM((R, _LANES), jnp.float32),
            pltpu.VMEM((_ZBLK, _LANES), jnp.float32),
            pltpu.SemaphoreType.DMA,
        ],
    )
    def k(h_hbm, src_hbm, dst_hbm, y_hbm,
          deg_sh, hp_sh, acc_sh, src_v, dst_v, rows, ones_v,
          degf, disf, hv, hpv, zerov, sem):
        c = lax.axis_index("c")
        s = lax.axis_index("s")
        w = c * _NS + s
        row0 = s * R

        fzero = jnp.zeros((_LANES,), jnp.float32)
        fone = jnp.full((_LANES,), 1.0, jnp.float32)
        half = jnp.full((_LANES,), 0.5, jnp.float32)
        three_half = jnp.full((_LANES,), 1.5, jnp.float32)
        magic = jnp.full((_LANES,), 0x5F3759DF, jnp.int32)
        one_i = jnp.full((_LANES,), 1, jnp.int32)

        def zfill_body(i, _):
            zerov[i] = fzero
            return 0
        lax.fori_loop(0, _ZBLK, zfill_body, 0)

        def ones_body(i, _):
            ones_v[pl.ds(i * _LANES, _LANES)] = fone
            return 0
        lax.fori_loop(0, 112 // _LANES, ones_body, 0)

        def degz_body(i, _):
            degf[pl.ds(i * _LANES, _LANES)] = fzero
            return 0
        lax.fori_loop(0, R // _LANES, degz_body, 0)
        pltpu.sync_copy(degf, deg_sh.at[pl.ds(row0, R)])

        def zero_acc(i, _):
            pltpu.sync_copy(zerov, acc_sh.at[pl.ds(row0 + i * _ZBLK, _ZBLK)])
            return 0
        lax.fori_loop(0, R // _ZBLK, zero_acc, 0)
        plsc.subcore_barrier()

        def deg_pass(dw):
            pltpu.sync_copy(dst_hbm.at[dw], dst_v)

            def deg_body(g, _):
                descs = [
                    pltpu.async_copy(
                        ones_v.at[pl.ds(0, _CHUNK)],
                        deg_sh.at[dst_v.at[_NBUF * g + b]], sem,
                        add=True)
                    for b in range(_NBUF)
                ]
                for dsc in descs:
                    dsc.wait()
                return 0
            lax.fori_loop(0, cw // _NBUF, deg_body, 0)

        deg_pass(2 * s)
        deg_pass(2 * s + 1)
        plsc.subcore_barrier()

        pltpu.sync_copy(deg_sh.at[pl.ds(row0, R)], degf)
        pltpu.sync_copy(h_hbm.at[pl.ds(row0, R)], hv)

        def rs_body(i, _):
            d = degf[pl.ds(i * _LANES, _LANES)] + fone
            bits = plsc.bitcast(d, jnp.int32)
            y = plsc.bitcast(
                magic - lax.shift_right_arithmetic(bits, one_i), jnp.float32)
            hd = half * d
            y = y * (three_half - hd * y * y)
            y = y * (three_half - hd * y * y)
            y = y * (three_half - hd * y * y)
            disf[pl.ds(i * _LANES, _LANES)] = y
            return 0
        lax.fori_loop(0, R // _LANES, rs_body, 0)

        def hp_body(i, _):
            dsp = plsc.load_gather(disf, [jnp.full((_LANES,), 1, jnp.int32) * i])
            hpv[i] = hv[i] * dsp
            return 0
        lax.fori_loop(0, R, hp_body, 0)
        pltpu.sync_copy(hpv, hp_sh.at[pl.ds(row0, R)])

        pltpu.sync_copy(src_hbm.at[w], src_v)
        pltpu.sync_copy(dst_hbm.at[w], dst_v)
        plsc.subcore_barrier()

        def edge_body(g, _):
            descs = [
                pltpu.async_copy(
                    hp_sh.at[src_v.at[_NBUF * g + b]], rows[b], sem)
                for b in range(_NBUF)
            ]
            for b in range(_NBUF):
                descs[b].wait()
                pltpu.sync_copy(
                    rows[b], acc_sh.at[dst_v.at[_NBUF * g + b]], add=True)
            return 0
        lax.fori_loop(0, cw // _NBUF, edge_body, 0)
        plsc.subcore_barrier()

        pltpu.sync_copy(acc_sh.at[pl.ds(row0, R)], hv)

        def y_body(i, _):
            dsp = plsc.load_gather(disf, [jnp.full((_LANES,), 1, jnp.int32) * i])
            hpv[i] = dsp * (hv[i] + half * hpv[i])
            return 0
        lax.fori_loop(0, R, y_body, 0)
        pltpu.sync_copy(hpv, y_hbm.at[c, pl.ds(row0, R)])

    return k(h_pad, src_w, dst_w)


def _tc_linear(x, w1, b1, *, n, n_pad, h):
    blk = n_pad // 2
    grid = n_pad // blk

    def body(x_ref, w_ref, b_ref, o_ref):
        i = pl.program_id(0)
        acc = jnp.dot(x_ref[...], w_ref[...],
                      preferred_element_type=jnp.float32) + b_ref[...]
        rows = i * blk + lax.broadcasted_iota(jnp.int32, (blk, h), 0)
        o_ref[...] = jnp.where(rows < n, acc, 0.0)

    d = x.shape[1]
    return pl.pallas_call(
        body,
        grid=(grid,),
        in_specs=[
            pl.BlockSpec((blk, d), lambda i: (i, 0)),
            pl.BlockSpec((d, h), lambda i: (0, 0)),
            pl.BlockSpec((1, h), lambda i: (0, 0)),
        ],
        out_specs=pl.BlockSpec((blk, h), lambda i: (i, 0)),
        out_shape=jax.ShapeDtypeStruct((n_pad, h), jnp.float32),
    )(x, w1, b1.reshape(1, h))


def _tc_head(y0, y1, w2, b2, *, n, h, c):
    blk = -(-n // 2 // 8) * 8
    grid = -(-n // blk)

    def body(a_ref, b_ref, w_ref, bias_ref, o_ref):
        z = jnp.maximum(a_ref[...] + b_ref[...], 0.0)
        logits = jnp.dot(z, w_ref[...],
                         preferred_element_type=jnp.float32) + bias_ref[...]
        m = jnp.max(logits, axis=1, keepdims=True)
        e = jnp.exp(logits - m)
        o_ref[...] = e / jnp.sum(e, axis=1, keepdims=True)

    rows = pl.BlockSpec((blk, h), lambda i: (i, 0))
    return pl.pallas_call(
        body,
        grid=(grid,),
        in_specs=[
            rows, rows,
            pl.BlockSpec((h, c), lambda i: (0, 0)),
            pl.BlockSpec((1, c), lambda i: (0, 0)),
        ],
        out_specs=pl.BlockSpec((blk, c), lambda i: (i, 0)),
        out_shape=jax.ShapeDtypeStruct((n, c), jnp.float32),
    )(y0, y1, w2, b2.reshape(1, c))


def kernel(x, edge_index, W1, b1, W2, b2):
    n, d = x.shape
    h = W1.shape[1]
    c = W2.shape[1]
    e = edge_index.shape[1]

    n_pad = -(-(n + 64) // 256) * 256
    junk = n_pad - n
    epw = _NC * _NS * _CHUNK * _NBUF
    e_pad = -(-e // epw) * epw
    cw = e_pad // (_NC * _NS * _CHUNK)

    h_pad = _tc_linear(x, W1, b1, n=n, n_pad=n_pad, h=h)

    src = edge_index[0]
    dst = edge_index[1]
    pad_cnt = e_pad - e
    if pad_cnt:
        pad_idx = n + jnp.arange(pad_cnt, dtype=jnp.int32) % junk
        src = jnp.concatenate([src, pad_idx])
        dst = jnp.concatenate([dst, pad_idx])
    src_w = src.reshape(_NC * _NS, cw, _CHUNK)
    dst_w = dst.reshape(_NC * _NS, cw, _CHUNK)

    y = _sc_gcn(h_pad, src_w, dst_w, n_pad=n_pad, cw=cw)
    return _tc_head(y[0], y[1], W2, b2, n=n, h=h, c=c)


# speedup vs baseline: 1.3598x; 1.0678x over previous
import functools

import jax
import jax.numpy as jnp
from jax import lax
from jax.experimental import pallas as pl
from jax.experimental.pallas import tpu as pltpu
from jax.experimental.pallas import tpu_sc as plsc

_NC = 2
_NS = 16
_LANES = 16
_CHUNK = 100
_NBUF = 10
_ZBLK = 64


_SC_PARAMS = pltpu.CompilerParams(use_tc_tiling_on_sc=False,
                                  needs_layout_passes=False)


def _sc_mesh():
    return plsc.VectorSubcoreMesh(
        core_axis_name="c", subcore_axis_name="s",
        num_cores=_NC, num_subcores=_NS)


def _sc_degree(dst_w, *, n_pad, cw):
    R = n_pad // _NS

    @functools.partial(
        pl.kernel,
        out_type=jax.ShapeDtypeStruct((_NC, n_pad), jnp.float32),
        mesh=_sc_mesh(),
        compiler_params=_SC_PARAMS,
        scratch_types=[
            pltpu.VMEM_SHARED((n_pad,), jnp.float32),
            pltpu.VMEM((cw, _CHUNK), jnp.int32),
            pltpu.VMEM((112,), jnp.float32),
            pltpu.VMEM((R,), jnp.float32),
            pltpu.SemaphoreType.DMA,
        ],
    )
    def k(dst_hbm, deg_hbm, deg_sh, dst_v, ones_v, degf, sem):
        c = lax.axis_index("c")
        s = lax.axis_index("s")
        w = c * _NS + s
        row0 = s * R

        pltpu.sync_copy(dst_hbm.at[w], dst_v)

        fzero = jnp.zeros((_LANES,), jnp.float32)
        fone = jnp.full((_LANES,), 1.0, jnp.float32)

        def ones_body(i, _):
            ones_v[pl.ds(i * _LANES, _LANES)] = fone
            return 0
        lax.fori_loop(0, 112 // _LANES, ones_body, 0)

        def zf_body(i, _):
            degf[pl.ds(i * _LANES, _LANES)] = fzero
            return 0
        lax.fori_loop(0, R // _LANES, zf_body, 0)
        pltpu.sync_copy(degf, deg_sh.at[pl.ds(row0, R)])
        plsc.subcore_barrier()

        def deg_body(g, _):
            descs = [
                pltpu.async_copy(
                    ones_v.at[pl.ds(0, _CHUNK)],
                    deg_sh.at[dst_v.at[_NBUF * g + b]], sem, add=True)
                for b in range(_NBUF)
            ]
            for dsc in descs:
                dsc.wait()
            return 0
        lax.fori_loop(0, cw // _NBUF, deg_body, 0)
        plsc.subcore_barrier()

        pltpu.sync_copy(deg_sh.at[pl.ds(row0, R)], deg_hbm.at[c, pl.ds(row0, R)])

    return k(dst_w)


def _sc_aggregate(h_pad, deg_pair, src_w, dst_w, *, n_pad, cw):
    R = n_pad // _NS

    @functools.partial(
        pl.kernel,
        out_type=jax.ShapeDtypeStruct((_NC, n_pad, _LANES), jnp.float32),
        mesh=_sc_mesh(),
        compiler_params=_SC_PARAMS,
        scratch_types=[
            pltpu.VMEM_SHARED((n_pad, _LANES), jnp.float32),
            pltpu.VMEM_SHARED((n_pad, _LANES), jnp.float32),
            pltpu.VMEM((cw, _CHUNK), jnp.int32),
            pltpu.VMEM((cw, _CHUNK), jnp.int32),
            [pltpu.VMEM((_CHUNK, _LANES), jnp.float32)
             for _ in range(_NBUF)],
            pltpu.VMEM((R,), jnp.float32),
            pltpu.VMEM((R,), jnp.float32),
            pltpu.VMEM((R,), jnp.float32),
            pltpu.VMEM((R, _LANES), jnp.float32),
            pltpu.VMEM((R, _LANES), jnp.float32),
            pltpu.VMEM((_ZBLK, _LANES), jnp.float32),
            pltpu.SemaphoreType.DMA,
            pltpu.SemaphoreType.DMA,
        ],
    )
    def k(h_hbm, deg_hbm, src_hbm, dst_hbm, y_hbm,
          hp_sh, acc_sh, src_v, dst_v, rows,
          d0f, d1f, disf, hv, hpv, zerov, gsem, ssem):
        c = lax.axis_index("c")
        s = lax.axis_index("s")
        w = c * _NS + s
        row0 = s * R

        pltpu.sync_copy(src_hbm.at[w], src_v)
        pltpu.sync_copy(dst_hbm.at[w], dst_v)
        pltpu.sync_copy(h_hbm.at[pl.ds(row0, R)], hv)
        pltpu.sync_copy(deg_hbm.at[0, pl.ds(row0, R)], d0f)
        pltpu.sync_copy(deg_hbm.at[1, pl.ds(row0, R)], d1f)

        fzero = jnp.zeros((_LANES,), jnp.float32)
        fone = jnp.full((_LANES,), 1.0, jnp.float32)
        half = jnp.full((_LANES,), 0.5, jnp.float32)
        three_half = jnp.full((_LANES,), 1.5, jnp.float32)
        magic = jnp.full((_LANES,), 0x5F3759DF, jnp.int32)
        one_i = jnp.full((_LANES,), 1, jnp.int32)

        def zfill_body(i, _):
            zerov[i] = fzero
            return 0
        lax.fori_loop(0, _ZBLK, zfill_body, 0)

        def zero_acc(i, _):
            pltpu.sync_copy(zerov, acc_sh.at[pl.ds(row0 + i * _ZBLK, _ZBLK)])
            return 0
        lax.fori_loop(0, R // _ZBLK, zero_acc, 0)

        def rs_body(i, _):
            sl = pl.ds(i * _LANES, _LANES)
            d = d0f[sl] + d1f[sl] + fone
            bits = plsc.bitcast(d, jnp.int32)
            y = plsc.bitcast(
                magic - lax.shift_right_arithmetic(bits, one_i), jnp.float32)
            hd = half * d
            y = y * (three_half - hd * y * y)
            y = y * (three_half - hd * y * y)
            y = y * (three_half - hd * y * y)
            disf[sl] = y
            return 0
        lax.fori_loop(0, R // _LANES, rs_body, 0)

        def hp_body(i, _):
            dsp = plsc.load_gather(disf, [one_i * i])
            hpv[i] = hv[i] * dsp
            return 0
        lax.fori_loop(0, R, hp_body, 0)
        pltpu.sync_copy(hpv, hp_sh.at[pl.ds(row0, R)])
        plsc.subcore_barrier()

        def edge_body(g, _):
            gds = [
                pltpu.async_copy(
                    hp_sh.at[src_v.at[_NBUF * g + b]], rows[b], gsem)
                for b in range(_NBUF)
            ]
            sds = []
            for b in range(_NBUF):
                gds[b].wait()
                sds.append(pltpu.async_copy(
                    rows[b], acc_sh.at[dst_v.at[_NBUF * g + b]], ssem,
                    add=True))
            for dsc in sds:
                dsc.wait()
            return 0
        lax.fori_loop(0, cw // _NBUF, edge_body, 0)
        plsc.subcore_barrier()

        pltpu.sync_copy(acc_sh.at[pl.ds(row0, R)], hv)

        def y_body(i, _):
            dsp = plsc.load_gather(disf, [one_i * i])
            hpv[i] = dsp * (hv[i] + half * hpv[i])
            return 0
        lax.fori_loop(0, R, y_body, 0)
        pltpu.sync_copy(hpv, y_hbm.at[c, pl.ds(row0, R)])

    return k(h_pad, deg_pair, src_w, dst_w)


def _tc_linear(x, w1, b1, *, n, n_pad, h):
    def body(x_ref, w_ref, b_ref, o_ref):
        acc = jnp.dot(x_ref[...], w_ref[...],
                      preferred_element_type=jnp.float32) + b_ref[...]
        rows = lax.broadcasted_iota(jnp.int32, (n_pad, h), 0)
        o_ref[...] = jnp.where(rows < n, acc, 0.0)

    d = x.shape[1]
    return pl.pallas_call(
        body,
        grid=(1,),
        in_specs=[
            pl.BlockSpec((n_pad, d), lambda i: (0, 0)),
            pl.BlockSpec((d, h), lambda i: (0, 0)),
            pl.BlockSpec((1, h), lambda i: (0, 0)),
        ],
        out_specs=pl.BlockSpec((n_pad, h), lambda i: (0, 0)),
        out_shape=jax.ShapeDtypeStruct((n_pad, h), jnp.float32),
    )(x, w1, b1.reshape(1, h))


def _tc_head(y0, y1, w2, b2, *, n, h, c):
    n8 = -(-n // 8) * 8

    def body(a_ref, b_ref, w_ref, bias_ref, o_ref):
        z = jnp.maximum(a_ref[...] + b_ref[...], 0.0)
        logits = jnp.dot(z, w_ref[...],
                         preferred_element_type=jnp.float32) + bias_ref[...]
        m = jnp.max(logits, axis=1, keepdims=True)
        e = jnp.exp(logits - m)
        o_ref[...] = e / jnp.sum(e, axis=1, keepdims=True)

    rows = pl.BlockSpec((n8, h), lambda i: (0, 0))
    return pl.pallas_call(
        body,
        grid=(1,),
        in_specs=[
            rows, rows,
            pl.BlockSpec((h, c), lambda i: (0, 0)),
            pl.BlockSpec((1, c), lambda i: (0, 0)),
        ],
        out_specs=pl.BlockSpec((n8, c), lambda i: (0, 0)),
        out_shape=jax.ShapeDtypeStruct((n, c), jnp.float32),
    )(y0, y1, w2, b2.reshape(1, c))


def kernel(x, edge_index, W1, b1, W2, b2):
    n, d = x.shape
    h = W1.shape[1]
    c = W2.shape[1]
    e = edge_index.shape[1]

    n_pad = -(-(n + 64) // 256) * 256
    junk = n_pad - n
    epw = _NC * _NS * _CHUNK * _NBUF
    e_pad = -(-e // epw) * epw
    cw = e_pad // (_NC * _NS * _CHUNK)

    h_pad = _tc_linear(x, W1, b1, n=n, n_pad=n_pad, h=h)

    src = edge_index[0]
    dst = edge_index[1]
    pad_cnt = e_pad - e
    if pad_cnt:
        pad_idx = n + jnp.arange(pad_cnt, dtype=jnp.int32) % junk
        src = jnp.concatenate([src, pad_idx])
        dst = jnp.concatenate([dst, pad_idx])
    src_w = src.reshape(_NC * _NS, cw, _CHUNK)
    dst_w = dst.reshape(_NC * _NS, cw, _CHUNK)

    deg_pair = _sc_degree(dst_w, n_pad=n_pad, cw=cw)
    y = _sc_aggregate(h_pad, deg_pair, src_w, dst_w, n_pad=n_pad, cw=cw)
    return _tc_head(y[0], y[1], W2, b2, n=n, h=h, c=c)


# speedup vs baseline: 1.3698x; 1.0073x over previous
import functools

import jax
import jax.numpy as jnp
from jax import lax
from jax.experimental import pallas as pl
from jax.experimental.pallas import tpu as pltpu
from jax.experimental.pallas import tpu_sc as plsc

_NC = 2
_NS = 16
_LANES = 16
_CHUNK = 125
_NBUF = 10
_ZBLK = 64


_SC_PARAMS = pltpu.CompilerParams(use_tc_tiling_on_sc=False,
                                  needs_layout_passes=False)


def _sc_mesh():
    return plsc.VectorSubcoreMesh(
        core_axis_name="c", subcore_axis_name="s",
        num_cores=_NC, num_subcores=_NS)


def _sc_degree(dst_w, *, n_pad, cw):
    R = n_pad // _NS

    @functools.partial(
        pl.kernel,
        out_type=jax.ShapeDtypeStruct((_NC, n_pad), jnp.float32),
        mesh=_sc_mesh(),
        compiler_params=_SC_PARAMS,
        scratch_types=[
            pltpu.VMEM_SHARED((n_pad,), jnp.float32),
            pltpu.VMEM((cw, _CHUNK), jnp.int32),
            pltpu.VMEM((128,), jnp.float32),
            pltpu.VMEM((R,), jnp.float32),
            pltpu.SemaphoreType.DMA,
        ],
    )
    def k(dst_hbm, deg_hbm, deg_sh, dst_v, ones_v, degf, sem):
        c = lax.axis_index("c")
        s = lax.axis_index("s")
        w = c * _NS + s
        row0 = s * R

        pltpu.sync_copy(dst_hbm.at[w], dst_v)

        fzero = jnp.zeros((_LANES,), jnp.float32)
        fone = jnp.full((_LANES,), 1.0, jnp.float32)

        def ones_body(i, _):
            ones_v[pl.ds(i * _LANES, _LANES)] = fone
            return 0
        lax.fori_loop(0, 128 // _LANES, ones_body, 0)

        def zf_body(i, _):
            degf[pl.ds(i * _LANES, _LANES)] = fzero
            return 0
        lax.fori_loop(0, R // _LANES, zf_body, 0)
        pltpu.sync_copy(degf, deg_sh.at[pl.ds(row0, R)])
        plsc.subcore_barrier()

        def deg_body(g, _):
            descs = [
                pltpu.async_copy(
                    ones_v.at[pl.ds(0, _CHUNK)],
                    deg_sh.at[dst_v.at[_NBUF * g + b]], sem, add=True)
                for b in range(_NBUF)
            ]
            for dsc in descs:
                dsc.wait()
            return 0
        lax.fori_loop(0, cw // _NBUF, deg_body, 0)
        plsc.subcore_barrier()

        pltpu.sync_copy(deg_sh.at[pl.ds(row0, R)], deg_hbm.at[c, pl.ds(row0, R)])

    return k(dst_w)


def _sc_aggregate(h_pad, deg_pair, src_w, dst_w, *, n_pad, cw):
    R = n_pad // _NS

    @functools.partial(
        pl.kernel,
        out_type=jax.ShapeDtypeStruct((_NC, n_pad, _LANES), jnp.float32),
        mesh=_sc_mesh(),
        compiler_params=_SC_PARAMS,
        scratch_types=[
            pltpu.VMEM_SHARED((n_pad, _LANES), jnp.float32),
            pltpu.VMEM_SHARED((n_pad, _LANES), jnp.float32),
            pltpu.VMEM((cw, _CHUNK), jnp.int32),
            pltpu.VMEM((cw, _CHUNK), jnp.int32),
            [pltpu.VMEM((_CHUNK, _LANES), jnp.float32)
             for _ in range(_NBUF)],
            pltpu.VMEM((R,), jnp.float32),
            pltpu.VMEM((R,), jnp.float32),
            pltpu.VMEM((R,), jnp.float32),
            pltpu.VMEM((R, _LANES), jnp.float32),
            pltpu.VMEM((R, _LANES), jnp.float32),
            pltpu.VMEM((_ZBLK, _LANES), jnp.float32),
            pltpu.SemaphoreType.DMA,
            pltpu.SemaphoreType.DMA,
        ],
    )
    def k(h_hbm, deg_hbm, src_hbm, dst_hbm, y_hbm,
          hp_sh, acc_sh, src_v, dst_v, rows,
          d0f, d1f, disf, hv, hpv, zerov, gsem, ssem):
        c = lax.axis_index("c")
        s = lax.axis_index("s")
        w = c * _NS + s
        row0 = s * R

        pltpu.sync_copy(src_hbm.at[w], src_v)
        pltpu.sync_copy(dst_hbm.at[w], dst_v)
        pltpu.sync_copy(h_hbm.at[pl.ds(row0, R)], hv)
        pltpu.sync_copy(deg_hbm.at[0, pl.ds(row0, R)], d0f)
        pltpu.sync_copy(deg_hbm.at[1, pl.ds(row0, R)], d1f)

        fzero = jnp.zeros((_LANES,), jnp.float32)
        fone = jnp.full((_LANES,), 1.0, jnp.float32)
        half = jnp.full((_LANES,), 0.5, jnp.float32)
        three_half = jnp.full((_LANES,), 1.5, jnp.float32)
        magic = jnp.full((_LANES,), 0x5F3759DF, jnp.int32)
        one_i = jnp.full((_LANES,), 1, jnp.int32)

        def zfill_body(i, _):
            zerov[i] = fzero
            return 0
        lax.fori_loop(0, _ZBLK, zfill_body, 0)

        def zero_acc(i, _):
            pltpu.sync_copy(zerov, acc_sh.at[pl.ds(row0 + i * _ZBLK, _ZBLK)])
            return 0
        lax.fori_loop(0, R // _ZBLK, zero_acc, 0)

        def rs_body(i, _):
            sl = pl.ds(i * _LANES, _LANES)
            d = d0f[sl] + d1f[sl] + fone
            bits = plsc.bitcast(d, jnp.int32)
            y = plsc.bitcast(
                magic - lax.shift_right_arithmetic(bits, one_i), jnp.float32)
            hd = half * d
            y = y * (three_half - hd * y * y)
            y = y * (three_half - hd * y * y)
            y = y * (three_half - hd * y * y)
            disf[sl] = y
            return 0
        lax.fori_loop(0, R // _LANES, rs_body, 0)

        def hp_body(i, _):
            dsp = plsc.load_gather(disf, [one_i * i])
            hpv[i] = hv[i] * dsp
            return 0
        lax.fori_loop(0, R, hp_body, 0)
        pltpu.sync_copy(hpv, hp_sh.at[pl.ds(row0, R)])
        plsc.subcore_barrier()

        def edge_body(g, _):
            gds = [
                pltpu.async_copy(
                    hp_sh.at[src_v.at[_NBUF * g + b]], rows[b], gsem)
                for b in range(_NBUF)
            ]
            sds = []
            for b in range(_NBUF):
                gds[b].wait()
                sds.append(pltpu.async_copy(
                    rows[b], acc_sh.at[dst_v.at[_NBUF * g + b]], ssem,
                    add=True))
            for dsc in sds:
                dsc.wait()
            return 0
        lax.fori_loop(0, cw // _NBUF, edge_body, 0)
        plsc.subcore_barrier()

        pltpu.sync_copy(acc_sh.at[pl.ds(row0, R)], hv)

        def y_body(i, _):
            dsp = plsc.load_gather(disf, [one_i * i])
            hpv[i] = dsp * (hv[i] + half * hpv[i])
            return 0
        lax.fori_loop(0, R, y_body, 0)
        pltpu.sync_copy(hpv, y_hbm.at[c, pl.ds(row0, R)])

    return k(h_pad, deg_pair, src_w, dst_w)


def _tc_linear(x, w1, b1, *, n, n_pad, h):
    def body(x_ref, w_ref, b_ref, o_ref):
        acc = jnp.dot(x_ref[...], w_ref[...],
                      preferred_element_type=jnp.float32) + b_ref[...]
        rows = lax.broadcasted_iota(jnp.int32, (n_pad, h), 0)
        o_ref[...] = jnp.where(rows < n, acc, 0.0)

    d = x.shape[1]
    return pl.pallas_call(
        body,
        grid=(1,),
        in_specs=[
            pl.BlockSpec((n_pad, d), lambda i: (0, 0)),
            pl.BlockSpec((d, h), lambda i: (0, 0)),
            pl.BlockSpec((1, h), lambda i: (0, 0)),
        ],
        out_specs=pl.BlockSpec((n_pad, h), lambda i: (0, 0)),
        out_shape=jax.ShapeDtypeStruct((n_pad, h), jnp.float32),
    )(x, w1, b1.reshape(1, h))


def _tc_head(y0, y1, w2, b2, *, n, h, c):
    n8 = -(-n // 8) * 8

    def body(a_ref, b_ref, w_ref, bias_ref, o_ref):
        z = jnp.maximum(a_ref[...] + b_ref[...], 0.0)
        logits = jnp.dot(z, w_ref[...],
                         preferred_element_type=jnp.float32) + bias_ref[...]
        m = jnp.max(logits, axis=1, keepdims=True)
        e = jnp.exp(logits - m)
        o_ref[...] = e / jnp.sum(e, axis=1, keepdims=True)

    rows = pl.BlockSpec((n8, h), lambda i: (0, 0))
    return pl.pallas_call(
        body,
        grid=(1,),
        in_specs=[
            rows, rows,
            pl.BlockSpec((h, c), lambda i: (0, 0)),
            pl.BlockSpec((1, c), lambda i: (0, 0)),
        ],
        out_specs=pl.BlockSpec((n8, c), lambda i: (0, 0)),
        out_shape=jax.ShapeDtypeStruct((n, c), jnp.float32),
    )(y0, y1, w2, b2.reshape(1, c))


def kernel(x, edge_index, W1, b1, W2, b2):
    n, d = x.shape
    h = W1.shape[1]
    c = W2.shape[1]
    e = edge_index.shape[1]

    n_pad = -(-(n + 64) // 256) * 256
    junk = n_pad - n
    epw = _NC * _NS * _CHUNK * _NBUF
    e_pad = -(-e // epw) * epw
    cw = e_pad // (_NC * _NS * _CHUNK)

    h_pad = _tc_linear(x, W1, b1, n=n, n_pad=n_pad, h=h)

    src = edge_index[0]
    dst = edge_index[1]
    pad_cnt = e_pad - e
    if pad_cnt:
        pad_idx = n + jnp.arange(pad_cnt, dtype=jnp.int32) % junk
        src = jnp.concatenate([src, pad_idx])
        dst = jnp.concatenate([dst, pad_idx])
    src_w = src.reshape(_NC * _NS, cw, _CHUNK)
    dst_w = dst.reshape(_NC * _NS, cw, _CHUNK)

    deg_pair = _sc_degree(dst_w, n_pad=n_pad, cw=cw)
    y = _sc_aggregate(h_pad, deg_pair, src_w, dst_w, n_pad=n_pad, cw=cw)
    return _tc_head(y[0], y[1], W2, b2, n=n, h=h, c=c)
